# baseline (speedup 1.0000x reference)
import jax
import jax.numpy as jnp
from jax import lax
from jax.experimental import pallas as pl
from jax.experimental.pallas import tpu as pltpu
from jax.experimental.pallas import tpu_sc as plsc

N = 3_200_000
NC = 2
NS = 16
L = 16
NW = NC * NS
P = N // NW
C = 4_000
NCHUNK = P // C
NV = C // L
U = 5


def _signed(v, t):
    bits = lax.bitcast_convert_type(v, jnp.int32) | (t << 31)
    return lax.bitcast_convert_type(bits, jnp.float32)


def _lane_allreduce(v):
    lanes = lax.iota(jnp.int32, L)
    dnums = lax.GatherDimensionNumbers(
        offset_dims=(), collapsed_slice_dims=(0,), start_index_map=(0,))
    for d in (1, 2, 4, 8):
        g = lax.gather(v, (lanes ^ d)[:, None], dnums, slice_sizes=(1,),
                       mode=lax.GatherScatterMode.PROMISE_IN_BOUNDS)
        v = v + g
    return v


def _body(s_hbm, t_hbm, out_hbm,
          stash, t0, t1, sf0, sf1, tf0, tf1, pvec_buf, p_buf, shared,
          sem0, sem1, osem):
    c = lax.axis_index("c")
    sid = lax.axis_index("s")
    base_own = (sid * NC + c) * P
    base_f = (sid * NC + (1 - c)) * P
    sems = (sem0, sem1)
    tbufs = (t0, t1)
    sfbufs = (sf0, sf1)
    tfbufs = (tf0, tf1)

    def start(ci):
        slot = ci % 2
        off_o = base_own + ci * C
        off_f = base_f + ci * C
        pltpu.async_copy(s_hbm.at[pl.ds(off_o, C)],
                         stash.at[pl.ds(ci * C, C)], sems[slot])
        pltpu.async_copy(t_hbm.at[pl.ds(off_o, C)], tbufs[slot], sems[slot])
        pltpu.async_copy(s_hbm.at[pl.ds(off_f, C)], sfbufs[slot], sems[slot])
        return pltpu.async_copy(
            t_hbm.at[pl.ds(off_f, C)], tfbufs[slot], sems[slot])

    h = start(0)
    z = jnp.zeros((L,), jnp.float32)
    accs = tuple((z, z) for _ in range(U))
    for ci in range(NCHUNK):
        for _ in range(4):
            h.wait()
        if ci + 1 < NCHUNK:
            h_next = start(ci + 1)
        slot = ci % 2
        tb, sfb, tfb = tbufs[slot], sfbufs[slot], tfbufs[slot]
        cbase = ci * C

        @plsc.parallel_loop(0, NV, step=U, carry=accs)
        def accs(i, carry):
            out = []
            for j in range(U):
                a_all, a_sgn = carry[j]
                k = pl.ds(cbase + (i + j) * L, L)
                v = jnp.exp(stash[k])
                e = _signed(v, tb[pl.ds((i + j) * L, L)])
                stash[k] = e
                vf = jnp.exp(sfb[pl.ds((i + j) * L, L)])
                ef = _signed(vf, tfb[pl.ds((i + j) * L, L)])
                out.append((a_all + v + vf, a_sgn + e + ef))
            return tuple(out)

        if ci + 1 < NCHUNK:
            h = h_next

    acc_all, acc_sgn = accs[0]
    for j in range(1, U):
        acc_all = acc_all + accs[j][0]
        acc_sgn = acc_sgn + accs[j][1]
    pvec_buf[pl.ds(0, L)] = acc_all
    pvec_buf[pl.ds(L, L)] = acc_sgn
    pltpu.sync_copy(pvec_buf, shared.at[pl.ds(sid * 2 * L, 2 * L)])
    plsc.subcore_barrier()
    pltpu.sync_copy(shared, p_buf)

    def red(i, carry):
        a_all, a_sgn = carry
        return (a_all + p_buf[pl.ds(i * 2 * L, L)],
                a_sgn + p_buf[pl.ds(i * 2 * L + L, L)])

    acc_all, acc_sgn = lax.fori_loop(0, NS, red, (z, z))
    r_all = _lane_allreduce(acc_all)
    r_sgn = _lane_allreduce(acc_sgn)
    inv0 = 2.0 / (r_all + r_sgn)
    inv1 = 2.0 / (r_all - r_sgn)

    ohs = []
    for ci in range(NCHUNK):
        cbase = ci * C

        @plsc.parallel_loop(0, NV, step=U)
        def _(i):
            for j in range(U):
                k = pl.ds(cbase + (i + j) * L, L)
                e = stash[k]
                stash[k] = jnp.abs(e) * jnp.where(e < 0.0, inv1, inv0)

        ohs.append(pltpu.async_copy(
            stash.at[pl.ds(cbase, C)],
            out_hbm.at[pl.ds(base_own + cbase, C)], osem))
    for hh in ohs:
        hh.wait()


def kernel(s, T):
    mesh = plsc.VectorSubcoreMesh(core_axis_name="c", subcore_axis_name="s")
    return pl.kernel(
        _body,
        mesh=mesh,
        out_type=jax.ShapeDtypeStruct((N,), jnp.float32),
        scratch_types=[
            pltpu.VMEM((P,), jnp.float32),
            pltpu.VMEM((C,), jnp.int32),
            pltpu.VMEM((C,), jnp.int32),
            pltpu.VMEM((C,), jnp.float32),
            pltpu.VMEM((C,), jnp.float32),
            pltpu.VMEM((C,), jnp.int32),
            pltpu.VMEM((C,), jnp.int32),
            pltpu.VMEM((2 * L,), jnp.float32),
            pltpu.VMEM((NS * 2 * L,), jnp.float32),
            pltpu.VMEM_SHARED((NS * 2 * L,), jnp.float32),
            pltpu.SemaphoreType.DMA,
            pltpu.SemaphoreType.DMA,
            pltpu.SemaphoreType.DMA,
        ],
    )(s, T)
